# initial kernel scaffold (unmeasured)
import jax
import jax.numpy as jnp
from jax import lax
from jax.experimental import pallas as pl
from jax.experimental.pallas import tpu as pltpu

N_DEV = 4
N_TOK = 256
D_IN = 128
D_OUT = 256
N_EXP = 8
EXP_PER_DEV = 2
ROWS = N_TOK // N_DEV


def kernel(x, router_W, route_idx, expert_W, shared_W):
    def body(x_ref, rw_ref, idx_ref, ew_ref, sw_ref, out_ref,
             send_ref, recv_ref, send_sems, recv_sems):
        p = lax.axis_index("i")
        left = lax.rem(p + (N_DEV - 1), N_DEV)
        right = lax.rem(p + 1, N_DEV)

        barrier_sem = pltpu.get_barrier_semaphore()
        for nbr in (left, right):
            pl.semaphore_signal(
                barrier_sem, inc=1,
                device_id=(nbr,), device_id_type=pl.DeviceIdType.MESH,
            )
        pl.semaphore_wait(barrier_sem, 2)

        xf = x_ref[:, :]
        scores = jnp.dot(xf, rw_ref[:, :], preferred_element_type=jnp.float32)
        s_max = jnp.max(scores, axis=-1, keepdims=True)
        ex = jnp.exp(scores - s_max)
        probs = ex / jnp.sum(ex, axis=-1, keepdims=True)

        ridx = idx_ref[:, :]
        col = lax.broadcasted_iota(jnp.int32, (N_TOK, N_EXP), 1)
        xb = xf.astype(jnp.bfloat16)

        partial = jnp.zeros((N_TOK, D_OUT), jnp.float32)
        for k in range(EXP_PER_DEV):
            eid = EXP_PER_DEV * p + k
            prob_e = jnp.sum(
                jnp.where(col == eid, probs, 0.0), axis=1, keepdims=True
            )
            gate = jnp.where(ridx == eid, prob_e, 0.0)
            y = jnp.dot(
                xb, ew_ref[k, :, :].astype(jnp.bfloat16),
                preferred_element_type=jnp.float32,
            )
            partial = partial + gate * y

        def chunk(c):
            return lax.dynamic_slice(partial, (c * ROWS, 0), (ROWS, D_OUT))

        acc = chunk(left)
        for h in range(N_DEV - 1):
            send_ref[h, :, :] = acc
            rdma = pltpu.make_async_remote_copy(
                src_ref=send_ref.at[h],
                dst_ref=recv_ref.at[h],
                send_sem=send_sems.at[h],
                recv_sem=recv_sems.at[h],
                device_id=(right,),
                device_id_type=pl.DeviceIdType.MESH,
            )
            rdma.start()
            rdma.wait()
            c_next = lax.rem(p + (N_DEV - 2) - h + N_DEV, N_DEV)
            acc = recv_ref[h, :, :] + chunk(c_next)

        x_mine = lax.dynamic_slice(xf, (p * ROWS, 0), (ROWS, D_IN))
        shared_mine = jnp.dot(
            x_mine.astype(jnp.bfloat16), sw_ref[:, :].astype(jnp.bfloat16),
            preferred_element_type=jnp.float32,
        )
        out_ref[:, :] = acc + shared_mine

    return pl.pallas_call(
        body,
        out_shape=jax.ShapeDtypeStruct((ROWS, D_OUT), jnp.float32),
        in_specs=[pl.BlockSpec(memory_space=pltpu.VMEM)] * 5,
        out_specs=pl.BlockSpec(memory_space=pltpu.VMEM),
        scratch_shapes=[
            pltpu.VMEM((N_DEV - 1, ROWS, D_OUT), jnp.float32),
            pltpu.VMEM((N_DEV - 1, ROWS, D_OUT), jnp.float32),
            pltpu.SemaphoreType.DMA((N_DEV - 1,)),
            pltpu.SemaphoreType.DMA((N_DEV - 1,)),
        ],
        compiler_params=pltpu.CompilerParams(collective_id=0),
    )(x, router_W, route_idx, expert_W, shared_W)


# baseline (device time: 14488 ns/iter reference)
import jax
import jax.numpy as jnp
from jax import lax
from jax.experimental import pallas as pl
from jax.experimental.pallas import tpu as pltpu

N_DEV = 4
N_TOK = 256
D_IN = 128
D_OUT = 256
N_EXP = 8
EXP_PER_DEV = 2
ROWS = N_TOK // N_DEV


def kernel(x, router_W, route_idx, expert_W, shared_W):
    def body(x_ref, rw_ref, idx_ref, ew_ref, sw_ref, out_ref,
             partial_ref, send_ref, recv_ref, send_sems, recv_sems):
        p = lax.axis_index("i")
        left = lax.rem(p + (N_DEV - 1), N_DEV)
        right = lax.rem(p + 1, N_DEV)

        barrier_sem = pltpu.get_barrier_semaphore()
        for nbr in (left, right):
            pl.semaphore_signal(
                barrier_sem, inc=1,
                device_id=(nbr,), device_id_type=pl.DeviceIdType.MESH,
            )
        pl.semaphore_wait(barrier_sem, 2)

        xf = x_ref[:, :]
        scores = jnp.dot(xf, rw_ref[:, :], preferred_element_type=jnp.float32)
        s_max = jnp.max(scores, axis=-1, keepdims=True)
        ex = jnp.exp(scores - s_max)
        probs = ex / jnp.sum(ex, axis=-1, keepdims=True)

        ridx = idx_ref[:, :]
        col = lax.broadcasted_iota(jnp.int32, (N_TOK, N_EXP), 1)
        xb = xf.astype(jnp.bfloat16)

        partial = jnp.zeros((N_TOK, D_OUT), jnp.float32)
        for k in range(EXP_PER_DEV):
            eid = EXP_PER_DEV * p + k
            prob_e = jnp.sum(
                jnp.where(col == eid, probs, 0.0), axis=1, keepdims=True
            )
            gate = jnp.where(ridx == eid, prob_e, 0.0)
            y = jnp.dot(
                xb, ew_ref[k, :, :].astype(jnp.bfloat16),
                preferred_element_type=jnp.float32,
            )
            partial = partial + gate * y
        partial_ref[:, :] = partial

        def chunk(c):
            return partial_ref[pl.ds(c * ROWS, ROWS), :]

        acc = chunk(left)
        for h in range(N_DEV - 1):
            send_ref[h, :, :] = acc
            rdma = pltpu.make_async_remote_copy(
                src_ref=send_ref.at[h],
                dst_ref=recv_ref.at[h],
                send_sem=send_sems.at[h],
                recv_sem=recv_sems.at[h],
                device_id=(right,),
                device_id_type=pl.DeviceIdType.MESH,
            )
            rdma.start()
            rdma.wait()
            c_next = lax.rem(p + (N_DEV - 2) - h + N_DEV, N_DEV)
            acc = recv_ref[h, :, :] + chunk(c_next)

        x_mine = x_ref[pl.ds(p * ROWS, ROWS), :]
        shared_mine = jnp.dot(
            x_mine.astype(jnp.bfloat16), sw_ref[:, :].astype(jnp.bfloat16),
            preferred_element_type=jnp.float32,
        )
        out_ref[:, :] = acc + shared_mine

    return pl.pallas_call(
        body,
        out_shape=jax.ShapeDtypeStruct((ROWS, D_OUT), jnp.float32),
        in_specs=[pl.BlockSpec(memory_space=pltpu.VMEM)] * 5,
        out_specs=pl.BlockSpec(memory_space=pltpu.VMEM),
        scratch_shapes=[
            pltpu.VMEM((N_TOK, D_OUT), jnp.float32),
            pltpu.VMEM((N_DEV - 1, ROWS, D_OUT), jnp.float32),
            pltpu.VMEM((N_DEV - 1, ROWS, D_OUT), jnp.float32),
            pltpu.SemaphoreType.DMA((N_DEV - 1,)),
            pltpu.SemaphoreType.DMA((N_DEV - 1,)),
        ],
        compiler_params=pltpu.CompilerParams(collective_id=0),
    )(x, router_W, route_idx, expert_W, shared_W)


# device time: 9688 ns/iter; 1.4955x vs baseline; 1.4955x over previous
import jax
import jax.numpy as jnp
from jax import lax
from jax.experimental import pallas as pl
from jax.experimental.pallas import tpu as pltpu

N_DEV = 4
N_TOK = 256
D_IN = 128
D_OUT = 256
N_EXP = 8
EXP_PER_DEV = 2
ROWS = N_TOK // N_DEV


def kernel(x, router_W, route_idx, expert_W, shared_W):
    def body(x_ref, rw_ref, idx_ref, ew_ref, sw_ref, out_ref,
             partial_ref, recv_ref, send_sems, recv_sems):
        p = lax.axis_index("i")

        barrier_sem = pltpu.get_barrier_semaphore()
        for off in range(1, N_DEV):
            pl.semaphore_signal(
                barrier_sem, inc=1,
                device_id=(lax.rem(p + off, N_DEV),),
                device_id_type=pl.DeviceIdType.MESH,
            )
        pl.semaphore_wait(barrier_sem, N_DEV - 1)

        xf = x_ref[:, :]
        scores = jnp.dot(xf, rw_ref[:, :], preferred_element_type=jnp.float32)
        s_max = jnp.max(scores, axis=-1, keepdims=True)
        ex = jnp.exp(scores - s_max)
        probs = ex / jnp.sum(ex, axis=-1, keepdims=True)

        ridx = idx_ref[:, :]
        col = lax.broadcasted_iota(jnp.int32, (N_TOK, N_EXP), 1)
        xb = xf.astype(jnp.bfloat16)

        partial = jnp.zeros((N_TOK, D_OUT), jnp.float32)
        for k in range(EXP_PER_DEV):
            eid = EXP_PER_DEV * p + k
            prob_e = jnp.sum(
                jnp.where(col == eid, probs, 0.0), axis=1, keepdims=True
            )
            gate = jnp.where(ridx == eid, prob_e, 0.0)
            y = jnp.dot(
                xb, ew_ref[k, :, :].astype(jnp.bfloat16),
                preferred_element_type=jnp.float32,
            )
            partial = partial + gate * y
        partial_ref[:, :] = partial.astype(jnp.bfloat16)

        rdmas = []
        for off in (2, 1, 3):
            d = lax.rem(p + off, N_DEV)
            j = off - 1
            rdma = pltpu.make_async_remote_copy(
                src_ref=partial_ref.at[pl.ds(d * ROWS, ROWS), :],
                dst_ref=recv_ref.at[j],
                send_sem=send_sems.at[j],
                recv_sem=recv_sems.at[j],
                device_id=(d,),
                device_id_type=pl.DeviceIdType.MESH,
            )
            rdma.start()
            rdmas.append(rdma)

        x_mine = x_ref[pl.ds(p * ROWS, ROWS), :]
        shared_mine = jnp.dot(
            x_mine.astype(jnp.bfloat16), sw_ref[:, :].astype(jnp.bfloat16),
            preferred_element_type=jnp.float32,
        )
        acc = shared_mine + partial_ref[pl.ds(p * ROWS, ROWS), :].astype(
            jnp.float32
        )

        for rdma in rdmas:
            rdma.wait_recv()
        for j in range(N_DEV - 1):
            acc = acc + recv_ref[j, :, :].astype(jnp.float32)
        out_ref[:, :] = acc

        for rdma in rdmas:
            rdma.wait_send()

    return pl.pallas_call(
        body,
        out_shape=jax.ShapeDtypeStruct((ROWS, D_OUT), jnp.float32),
        in_specs=[pl.BlockSpec(memory_space=pltpu.VMEM)] * 5,
        out_specs=pl.BlockSpec(memory_space=pltpu.VMEM),
        scratch_shapes=[
            pltpu.VMEM((N_TOK, D_OUT), jnp.bfloat16),
            pltpu.VMEM((N_DEV - 1, ROWS, D_OUT), jnp.bfloat16),
            pltpu.SemaphoreType.DMA((N_DEV - 1,)),
            pltpu.SemaphoreType.DMA((N_DEV - 1,)),
        ],
        compiler_params=pltpu.CompilerParams(collective_id=0),
    )(x, router_W, route_idx, expert_W, shared_W)


# device time: 9463 ns/iter; 1.5310x vs baseline; 1.0238x over previous
import jax
import jax.numpy as jnp
from jax import lax
from jax.experimental import pallas as pl
from jax.experimental.pallas import tpu as pltpu

N_DEV = 4
N_TOK = 256
D_IN = 128
D_OUT = 256
N_EXP = 8
EXP_PER_DEV = 2
ROWS = N_TOK // N_DEV


def kernel(x, router_W, route_idx, expert_W, shared_W):
    def body(x_ref, rw_ref, idx_ref, ew_ref, sw_ref, out_ref,
             xg_ref, send_ref, recv_ref, send_sems, recv_sems):
        p = lax.axis_index("i")

        barrier_sem = pltpu.get_barrier_semaphore()
        for off in range(1, N_DEV):
            pl.semaphore_signal(
                barrier_sem, inc=1,
                device_id=(lax.rem(p + off, N_DEV),),
                device_id_type=pl.DeviceIdType.MESH,
            )

        xf = x_ref[:, :]
        scores = jnp.dot(xf, rw_ref[:, :], preferred_element_type=jnp.float32)
        s_max = jnp.max(scores, axis=-1, keepdims=True)
        ex = jnp.exp(scores - s_max)
        probs = ex / jnp.sum(ex, axis=-1, keepdims=True)

        ridx = idx_ref[:, :]
        col = lax.broadcasted_iota(jnp.int32, (N_TOK, N_EXP), 1)
        xb = xf.astype(jnp.bfloat16)

        gated = []
        for k in range(EXP_PER_DEV):
            eid = EXP_PER_DEV * p + k
            prob_e = jnp.sum(
                jnp.where(col == eid, probs, 0.0), axis=1, keepdims=True
            )
            gate = jnp.where(ridx == eid, prob_e, 0.0)
            gated.append(xb * gate.astype(jnp.bfloat16))
        xg_ref[:, :] = jnp.concatenate(gated, axis=1)

        ew = ew_ref[:, :, :].astype(jnp.bfloat16).reshape(
            EXP_PER_DEV * D_IN, D_OUT
        )

        pl.semaphore_wait(barrier_sem, N_DEV - 1)

        rdmas = []
        for off in (2, 1, 3):
            d = lax.rem(p + off, N_DEV)
            j = off - 1
            y = jnp.dot(
                xg_ref[pl.ds(d * ROWS, ROWS), :], ew,
                preferred_element_type=jnp.float32,
            )
            send_ref[j, :, :] = y.astype(jnp.bfloat16)
            rdma = pltpu.make_async_remote_copy(
                src_ref=send_ref.at[j],
                dst_ref=recv_ref.at[j],
                send_sem=send_sems.at[j],
                recv_sem=recv_sems.at[j],
                device_id=(d,),
                device_id_type=pl.DeviceIdType.MESH,
            )
            rdma.start()
            rdmas.append(rdma)

        y_mine = jnp.dot(
            xg_ref[pl.ds(p * ROWS, ROWS), :], ew,
            preferred_element_type=jnp.float32,
        )
        x_mine = x_ref[pl.ds(p * ROWS, ROWS), :]
        shared_mine = jnp.dot(
            x_mine.astype(jnp.bfloat16), sw_ref[:, :].astype(jnp.bfloat16),
            preferred_element_type=jnp.float32,
        )
        acc = shared_mine + y_mine

        for rdma in rdmas:
            rdma.wait_recv()
        for j in range(N_DEV - 1):
            acc = acc + recv_ref[j, :, :].astype(jnp.float32)
        out_ref[:, :] = acc

        for rdma in rdmas:
            rdma.wait_send()

    return pl.pallas_call(
        body,
        out_shape=jax.ShapeDtypeStruct((ROWS, D_OUT), jnp.float32),
        in_specs=[pl.BlockSpec(memory_space=pltpu.VMEM)] * 5,
        out_specs=pl.BlockSpec(memory_space=pltpu.VMEM),
        scratch_shapes=[
            pltpu.VMEM((N_TOK, EXP_PER_DEV * D_IN), jnp.bfloat16),
            pltpu.VMEM((N_DEV - 1, ROWS, D_OUT), jnp.bfloat16),
            pltpu.VMEM((N_DEV - 1, ROWS, D_OUT), jnp.bfloat16),
            pltpu.SemaphoreType.DMA((N_DEV - 1,)),
            pltpu.SemaphoreType.DMA((N_DEV - 1,)),
        ],
        compiler_params=pltpu.CompilerParams(collective_id=0),
    )(x, router_W, route_idx, expert_W, shared_W)
